# Initial kernel scaffold; baseline (speedup 1.0000x reference)
import functools

import jax
import jax.numpy as jnp
from jax import lax
from jax.experimental import pallas as pl
from jax.experimental.pallas import tpu as pltpu
from jax.experimental.pallas import tpu_sc as plsc

DIM = 300
NC = 2
NS = 16
NW = NC * NS
C = 128

_mesh = plsc.VectorSubcoreMesh(core_axis_name="c", subcore_axis_name="s")


def _make_gather(n_total: int):
    bpw = n_total // NW
    nchunk = bpw // C

    @functools.partial(
        pl.kernel,
        out_type=jax.ShapeDtypeStruct((n_total, DIM), jnp.float32),
        mesh=_mesh,
        scratch_types=[
            pltpu.VMEM((nchunk, C), jnp.int32),
            pltpu.VMEM((2, C, DIM), jnp.float32),
            pltpu.SemaphoreType.DMA,
            pltpu.SemaphoreType.DMA,
            pltpu.SemaphoreType.DMA,
            pltpu.SemaphoreType.DMA,
        ],
    )
    def gather_kernel(idx_hbm, table_hbm, out_hbm, idx_v, rows_v, g0, g1, o0, o1):
        wid = lax.axis_index("s") * NC + lax.axis_index("c")
        pltpu.sync_copy(idx_hbm.at[pl.ds(wid * nchunk, nchunk)], idx_v)
        out_base = wid * bpw

        gsem = (g0, g1)
        osem = (o0, o1)

        def gather_start(j, b):
            pltpu.async_copy(table_hbm.at[idx_v.at[j]], rows_v.at[b], gsem[b])

        def gather_wait(b):
            pltpu.make_async_copy(table_hbm.at[idx_v.at[0]], rows_v.at[b],
                                  gsem[b]).wait()

        def write_start(j, b):
            pltpu.async_copy(rows_v.at[b], out_hbm.at[pl.ds(out_base + j * C, C)],
                             osem[b])

        def write_wait(b):
            pltpu.make_async_copy(rows_v.at[b],
                                  out_hbm.at[pl.ds(out_base, C)], osem[b]).wait()

        gather_start(0, 0)
        gather_start(1, 1)

        def step(t, carry):
            j = t * 2

            gather_wait(0)
            write_start(j, 0)

            @pl.when(j + 2 < nchunk)
            def _():
                write_wait(0)
                gather_start(j + 2, 0)

            gather_wait(1)
            write_start(j + 1, 1)

            @pl.when(j + 3 < nchunk)
            def _():
                write_wait(1)
                gather_start(j + 3, 1)

            return carry

        lax.fori_loop(0, nchunk // 2, step, 0)

        write_wait(0)
        write_wait(1)

    return gather_kernel


_gather = _make_gather(4096 * 50)


def kernel(batch, table):
    b, l = batch.shape[1], batch.shape[2]
    idx = batch.reshape(-1).astype(jnp.int32)
    idx2 = idx.reshape(-1, C)
    out = _gather(idx2, table)
    return out.reshape(b, l, DIM)


# speedup vs baseline: 3.3963x; 3.3963x over previous
import functools

import jax
import jax.numpy as jnp
from jax import lax
from jax.experimental import pallas as pl
from jax.experimental.pallas import tpu as pltpu
from jax.experimental.pallas import tpu_sc as plsc

DIM = 300
DIMP = 304
NC = 2
NS = 16
NW = NC * NS
C = 128

_mesh = plsc.VectorSubcoreMesh(core_axis_name="c", subcore_axis_name="s")


def _make_gather(n_total: int):
    bpw = n_total // NW
    nchunk = bpw // C

    @functools.partial(
        pl.kernel,
        out_type=jax.ShapeDtypeStruct((n_total, DIMP), jnp.float32),
        mesh=_mesh,
        scratch_types=[
            pltpu.VMEM((bpw,), jnp.int32),
            pltpu.VMEM((2, C, DIMP), jnp.float32),
            pltpu.SemaphoreType.DMA,
            pltpu.SemaphoreType.DMA,
            pltpu.SemaphoreType.DMA,
            pltpu.SemaphoreType.DMA,
        ],
        compiler_params=pltpu.CompilerParams(use_tc_tiling_on_sc=False),
    )
    def gather_kernel(idx_hbm, table_hbm, out_hbm, idx_v, rows_v, g0, g1, o0, o1):
        wid = lax.axis_index("s") * NC + lax.axis_index("c")
        out_base = wid * bpw
        pltpu.sync_copy(idx_hbm.at[pl.ds(out_base, bpw)], idx_v)

        gsem = (g0, g1)
        osem = (o0, o1)

        def gather_start(j, b):
            pltpu.async_copy(table_hbm.at[idx_v.at[pl.ds(j * C, C)]],
                             rows_v.at[b], gsem[b])

        def gather_wait(b):
            pltpu.make_async_copy(table_hbm.at[idx_v.at[pl.ds(0, C)]],
                                  rows_v.at[b], gsem[b]).wait()

        def write_start(j, b):
            pltpu.async_copy(rows_v.at[b],
                             out_hbm.at[pl.ds(out_base + j * C, C)], osem[b])

        def write_wait(b):
            pltpu.make_async_copy(rows_v.at[b],
                                  out_hbm.at[pl.ds(out_base, C)], osem[b]).wait()

        gather_start(0, 0)
        gather_start(1, 1)

        def step(t, carry):
            j = t * 2

            gather_wait(0)
            write_start(j, 0)

            @pl.when(j + 2 < nchunk)
            def _():
                write_wait(0)
                gather_start(j + 2, 0)

            gather_wait(1)
            write_start(j + 1, 1)

            @pl.when(j + 3 < nchunk)
            def _():
                write_wait(1)
                gather_start(j + 3, 1)

            return carry

        lax.fori_loop(0, nchunk // 2, step, 0)

        write_wait(0)
        write_wait(1)

    return gather_kernel


_gather = _make_gather(4096 * 50)


def kernel(batch, table):
    b, l = batch.shape[1], batch.shape[2]
    idx = batch.reshape(-1).astype(jnp.int32)
    table_p = jnp.pad(table, ((0, 0), (0, DIMP - DIM)))
    out = _gather(idx, table_p)
    return out[:, :DIM].reshape(b, l, DIM)
